# Initial kernel scaffold; baseline (speedup 1.0000x reference)
import jax
import jax.numpy as jnp
from jax.experimental import pallas as pl


def kernel(mem, idx, val):
    raise NotImplementedError("write your pallas kernel here")


# speedup vs baseline: 1.7226x; 1.7226x over previous
import functools

import jax
import jax.numpy as jnp
from jax import lax
from jax.experimental import pallas as pl
from jax.experimental.pallas import tpu as pltpu
from jax.experimental.pallas import tpu_sc as plsc

TEMP = 0.05
SMOOTH_WEIGHT = 0.2

_NC = 2
_NS = 16
_NW = _NC * _NS
_L = 16


def _vrsqrt(x):
    i = plsc.bitcast(x, jnp.int32)
    i = jnp.int32(0x5F3759DF) - lax.shift_right_arithmetic(i, 1)
    y = plsc.bitcast(i, jnp.float32)
    for _ in range(3):
        y = y * (1.5 - 0.5 * x * y * y)
    return y


def _make_sc_update(M, D, B):
    b_per_w = B // _NW
    nch = D // _L
    mesh = plsc.VectorSubcoreMesh(core_axis_name="c", subcore_axis_name="s")

    @functools.partial(
        pl.kernel,
        out_type=jax.ShapeDtypeStruct((B, D), jnp.float32),
        mesh=mesh,
        scratch_types=[
            pltpu.VMEM((b_per_w,), jnp.int32),
            pltpu.VMEM((b_per_w, D), jnp.float32),
            pltpu.VMEM((b_per_w, D), jnp.float32),
            pltpu.VMEM((b_per_w, D), jnp.float32),
            pltpu.SemaphoreType.DMA,
        ],
        compiler_params=pltpu.CompilerParams(needs_layout_passes=False),
    )
    def sc_update(mem_hbm, idx_hbm, val_hbm, upd_hbm, idx_v, g_v, v_v, u_v, sem):
        wid = lax.axis_index("s") * _NC + lax.axis_index("c")
        base = wid * b_per_w
        pltpu.sync_copy(idx_hbm.at[pl.ds(base, b_per_w)], idx_v)
        pltpu.async_copy(mem_hbm.at[idx_v], g_v, sem).wait()
        pltpu.sync_copy(val_hbm.at[pl.ds(base, b_per_w)], v_v)

        def row(i, carry):
            vs = []
            acc = jnp.zeros((_L,), jnp.float32)
            for c in range(nch):
                vc = v_v[i, pl.ds(c * _L, _L)]
                vs.append(vc)
                acc = acc + vc * vc
            rv = _vrsqrt(lax.broadcast_in_dim(jnp.sum(acc, axis=0), (_L,), ()))
            us = []
            accu = jnp.zeros((_L,), jnp.float32)
            for c in range(nch):
                gc = g_v[i, pl.ds(c * _L, _L)]
                uc = SMOOTH_WEIGHT * gc + (1.0 - SMOOTH_WEIGHT) * (vs[c] * rv)
                us.append(uc)
                accu = accu + uc * uc
            ru = _vrsqrt(lax.broadcast_in_dim(jnp.sum(accu, axis=0), (_L,), ()))
            for c in range(nch):
                u_v[i, pl.ds(c * _L, _L)] = us[c] * ru
            return carry

        lax.fori_loop(0, b_per_w, row, 0)
        pltpu.sync_copy(u_v, upd_hbm.at[pl.ds(base, b_per_w)])

    return sc_update


def _make_tc_loss_scatter(M, D, B, Mb):
    K = M // Mb
    inv_temp = 1.0 / TEMP

    def body(idx_smem, idx2d, mem_ref, val_ref, upd_ref,
             loss_ref, out_any, valn, m_ref, s_ref, t_ref, sem):
        k = pl.program_id(0)

        @pl.when(k == 0)
        def _init():
            v = val_ref[...]
            nv = jnp.sqrt(jnp.sum(v * v, axis=1, keepdims=True))
            valn[...] = v / (nv + 1e-12)
            m_ref[...] = jnp.full((B, 1), -1e30, jnp.float32)
            s_ref[...] = jnp.zeros((B, 1), jnp.float32)
            t_ref[...] = jnp.zeros((B, 1), jnp.float32)
            loss_ref[...] = jnp.zeros((8, 128), jnp.float32)

        g = mem_ref[...]
        nrm = jnp.sqrt(jnp.sum(g * g, axis=1, keepdims=True))
        gn = g / (nrm + 1e-12)
        logits = lax.dot_general(
            valn[...], gn, (((1,), (1,)), ((), ())),
            preferred_element_type=jnp.float32) * inv_temp
        bm = jnp.max(logits, axis=1, keepdims=True)
        mo = m_ref[...]
        mn = jnp.maximum(mo, bm)
        s_ref[...] = (s_ref[...] * jnp.exp(mo - mn)
                      + jnp.sum(jnp.exp(logits - mn), axis=1, keepdims=True))
        m_ref[...] = mn
        cols = k * Mb + lax.broadcasted_iota(jnp.int32, (B, Mb), 1)
        hit = cols == idx2d[...]
        t_ref[...] += jnp.sum(jnp.where(hit, logits, 0.0), axis=1, keepdims=True)

        @pl.when(k == K - 1)
        def _fin():
            lse = m_ref[...] + jnp.log(s_ref[...])
            loss = jnp.mean(lse - t_ref[...])
            loss_ref[...] = jnp.full((8, 128), loss, jnp.float32)
            nchunk = 8
            ch = B // nchunk

            def chunk(c, _):
                def fire(j, _):
                    i = c * ch + j
                    r = idx_smem[i]
                    pltpu.make_async_copy(
                        upd_ref.at[pl.ds(i, 1), :],
                        out_any.at[pl.ds(r, 1), :], sem).start()
                    return 0

                lax.fori_loop(0, ch, fire, 0)

                def drain(j, _):
                    pltpu.make_async_copy(
                        upd_ref.at[pl.ds(0, 1), :],
                        out_any.at[pl.ds(0, 1), :], sem).wait()
                    return 0

                lax.fori_loop(0, ch, drain, 0)
                return 0

            lax.fori_loop(0, nchunk, chunk, 0)

    return pl.pallas_call(
        body,
        grid=(K,),
        in_specs=[
            pl.BlockSpec(memory_space=pltpu.SMEM),
            pl.BlockSpec((B, 1), lambda k: (0, 0)),
            pl.BlockSpec((Mb, D), lambda k: (k, 0)),
            pl.BlockSpec((B, D), lambda k: (0, 0)),
            pl.BlockSpec((B, D), lambda k: (0, 0)),
        ],
        out_specs=[
            pl.BlockSpec((8, 128), lambda k: (0, 0)),
            pl.BlockSpec(memory_space=pl.ANY),
        ],
        out_shape=[
            jax.ShapeDtypeStruct((8, 128), jnp.float32),
            jax.ShapeDtypeStruct((M, D), jnp.float32),
        ],
        scratch_shapes=[
            pltpu.VMEM((B, D), jnp.float32),
            pltpu.VMEM((B, 1), jnp.float32),
            pltpu.VMEM((B, 1), jnp.float32),
            pltpu.VMEM((B, 1), jnp.float32),
            pltpu.SemaphoreType.DMA,
        ],
        input_output_aliases={2: 1},
        compiler_params=pltpu.CompilerParams(
            dimension_semantics=("arbitrary",)),
    )


def kernel(mem, idx, val):
    M, D = mem.shape
    B = val.shape[0]
    upd = _make_sc_update(M, D, B)(mem, idx, val)
    loss8, mem_new = _make_tc_loss_scatter(M, D, B, 1000)(
        idx, idx.reshape(B, 1), mem, val, upd)
    return loss8[0, 0], mem_new


# speedup vs baseline: 3.8662x; 2.2444x over previous
import functools

import jax
import jax.numpy as jnp
from jax import lax
from jax.experimental import pallas as pl
from jax.experimental.pallas import tpu as pltpu
from jax.experimental.pallas import tpu_sc as plsc

TEMP = 0.05
SMOOTH_WEIGHT = 0.2

_NC = 2
_NS = 16
_NW = _NC * _NS
_L = 16


def _vrsqrt(x):
    i = plsc.bitcast(x, jnp.int32)
    i = jnp.int32(0x5F3759DF) - lax.shift_right_arithmetic(i, 1)
    y = plsc.bitcast(i, jnp.float32)
    for _ in range(3):
        y = y * (1.5 - 0.5 * x * y * y)
    return y


def _make_sc_update(M, D, B):
    b_per_w = B // _NW
    nch = D // _L
    mesh = plsc.VectorSubcoreMesh(core_axis_name="c", subcore_axis_name="s")

    @functools.partial(
        pl.kernel,
        out_type=(
            jax.ShapeDtypeStruct((B, D), jnp.float32),
            jax.ShapeDtypeStruct((B,), jnp.float32),
        ),
        mesh=mesh,
        scratch_types=[
            pltpu.VMEM((b_per_w,), jnp.int32),
            pltpu.VMEM((b_per_w, D), jnp.float32),
            pltpu.VMEM((b_per_w, D), jnp.float32),
            pltpu.VMEM((b_per_w, D), jnp.float32),
            pltpu.VMEM((b_per_w,), jnp.float32),
            pltpu.SemaphoreType.DMA,
        ],
        compiler_params=pltpu.CompilerParams(needs_layout_passes=False),
    )
    def sc_update(mem_hbm, idx_hbm, val_hbm, upd_hbm, t_hbm,
                  idx_v, g_v, v_v, u_v, t_v, sem):
        wid = lax.axis_index("s") * _NC + lax.axis_index("c")
        base = wid * b_per_w
        pltpu.sync_copy(idx_hbm.at[pl.ds(base, b_per_w)], idx_v)
        pltpu.async_copy(mem_hbm.at[idx_v], g_v, sem).wait()
        pltpu.sync_copy(val_hbm.at[pl.ds(base, b_per_w)], v_v)

        lane = lax.broadcasted_iota(jnp.int32, (_L,), 0)

        for grp in range(b_per_w // _L):
            def row(j, tacc, _grp=grp):
                i = _grp * _L + j
                vs, gs = [], []
                acc_vv = jnp.zeros((_L,), jnp.float32)
                acc_gg = jnp.zeros((_L,), jnp.float32)
                acc_gv = jnp.zeros((_L,), jnp.float32)
                for c in range(nch):
                    vc = v_v[i, pl.ds(c * _L, _L)]
                    gc = g_v[i, pl.ds(c * _L, _L)]
                    vs.append(vc)
                    gs.append(gc)
                    acc_vv = acc_vv + vc * vc
                    acc_gg = acc_gg + gc * gc
                    acc_gv = acc_gv + gc * vc
                rv = _vrsqrt(
                    lax.broadcast_in_dim(jnp.sum(acc_vv, axis=0), (_L,), ()))
                rg = _vrsqrt(
                    lax.broadcast_in_dim(jnp.sum(acc_gg, axis=0), (_L,), ()))
                t = jnp.sum(acc_gv * rv * rg * (1.0 / TEMP), axis=0)
                tacc = jnp.where(lane == j,
                                 lax.broadcast_in_dim(t, (_L,), ()), tacc)
                us = []
                acc_uu = jnp.zeros((_L,), jnp.float32)
                for c in range(nch):
                    uc = (SMOOTH_WEIGHT * gs[c]
                          + (1.0 - SMOOTH_WEIGHT) * (vs[c] * rv))
                    us.append(uc)
                    acc_uu = acc_uu + uc * uc
                ru = _vrsqrt(
                    lax.broadcast_in_dim(jnp.sum(acc_uu, axis=0), (_L,), ()))
                for c in range(nch):
                    u_v[i, pl.ds(c * _L, _L)] = us[c] * ru
                return tacc

            t_v[pl.ds(grp * _L, _L)] = lax.fori_loop(
                0, _L, row, jnp.zeros((_L,), jnp.float32))
        pltpu.sync_copy(u_v, upd_hbm.at[pl.ds(base, b_per_w)])
        pltpu.sync_copy(t_v, t_hbm.at[pl.ds(base, b_per_w)])

    return sc_update


def _make_tc_loss_scatter(M, D, B, Mb):
    K = M // Mb
    inv_temp = 1.0 / TEMP

    def body(idx_smem, mem_ref, val_ref, upd_ref, t_ref,
             loss_ref, out_any, valn, s_ref, sem):
        k = pl.program_id(0)

        @pl.when(k == 0)
        def _init():
            v = val_ref[...]
            nv = jnp.sqrt(jnp.sum(v * v, axis=1, keepdims=True))
            valn[...] = (v * (inv_temp / (nv + 1e-12))).astype(jnp.bfloat16)
            s_ref[...] = jnp.zeros((B, 1), jnp.float32)
            loss_ref[...] = jnp.zeros((8, 128), jnp.float32)

        g = mem_ref[...]
        nrm = jnp.sqrt(jnp.sum(g * g, axis=1, keepdims=True))
        gn = (g * (1.0 / (nrm + 1e-12))).astype(jnp.bfloat16)
        logits = lax.dot_general(
            valn[...], gn, (((1,), (1,)), ((), ())),
            preferred_element_type=jnp.float32)
        s_ref[...] += jnp.sum(jnp.exp(logits), axis=1, keepdims=True)

        @pl.when(k == K - 1)
        def _fin():
            lse = jnp.log(s_ref[...])
            loss = jnp.mean(lse - t_ref[...])
            loss_ref[...] = jnp.full((8, 128), loss, jnp.float32)
            nchunk = 8
            ch = B // nchunk

            def chunk(c, _):
                def fire(j, _):
                    i = c * ch + j
                    r = idx_smem[i]
                    pltpu.make_async_copy(
                        upd_ref.at[pl.ds(i, 1), :],
                        out_any.at[pl.ds(r, 1), :], sem).start()
                    return 0

                lax.fori_loop(0, ch, fire, 0)

                def drain(j, _):
                    pltpu.make_async_copy(
                        upd_ref.at[pl.ds(0, 1), :],
                        out_any.at[pl.ds(0, 1), :], sem).wait()
                    return 0

                lax.fori_loop(0, ch, drain, 0)
                return 0

            lax.fori_loop(0, nchunk, chunk, 0)

    return pl.pallas_call(
        body,
        grid=(K,),
        in_specs=[
            pl.BlockSpec(memory_space=pltpu.SMEM),
            pl.BlockSpec((Mb, D), lambda k: (k, 0)),
            pl.BlockSpec((B, D), lambda k: (0, 0)),
            pl.BlockSpec((B, D), lambda k: (0, 0)),
            pl.BlockSpec((B, 1), lambda k: (0, 0)),
        ],
        out_specs=[
            pl.BlockSpec((8, 128), lambda k: (0, 0)),
            pl.BlockSpec(memory_space=pl.ANY),
        ],
        out_shape=[
            jax.ShapeDtypeStruct((8, 128), jnp.float32),
            jax.ShapeDtypeStruct((M, D), jnp.float32),
        ],
        scratch_shapes=[
            pltpu.VMEM((B, D), jnp.bfloat16),
            pltpu.VMEM((B, 1), jnp.float32),
            pltpu.SemaphoreType.DMA,
        ],
        input_output_aliases={1: 1},
        compiler_params=pltpu.CompilerParams(
            dimension_semantics=("arbitrary",)),
    )


def kernel(mem, idx, val):
    M, D = mem.shape
    B = val.shape[0]
    upd, t = _make_sc_update(M, D, B)(mem, idx, val)
    loss8, mem_new = _make_tc_loss_scatter(M, D, B, 2000)(
        idx, mem, val, upd, t.reshape(B, 1))
    return loss8[0, 0], mem_new


# speedup vs baseline: 4.9110x; 1.2702x over previous
import functools
import math

import jax
import jax.numpy as jnp
from jax import lax
from jax.experimental import pallas as pl
from jax.experimental.pallas import tpu as pltpu
from jax.experimental.pallas import tpu_sc as plsc

TEMP = 0.05
SMOOTH_WEIGHT = 0.2

_NC = 2
_NS = 16
_NW = _NC * _NS
_L = 16


def _vrsqrt(x):
    i = plsc.bitcast(x, jnp.int32)
    i = jnp.int32(0x5F3759DF) - lax.shift_right_arithmetic(i, 1)
    y = plsc.bitcast(i, jnp.float32)
    for _ in range(3):
        y = y * (1.5 - 0.5 * x * y * y)
    return y


def _make_sc_update(M, D, B):
    b_per_w = B // _NW
    nch = D // _L
    mesh = plsc.VectorSubcoreMesh(core_axis_name="c", subcore_axis_name="s")

    @functools.partial(
        pl.kernel,
        out_type=(
            jax.ShapeDtypeStruct((B, D), jnp.float32),
            jax.ShapeDtypeStruct((B,), jnp.float32),
        ),
        mesh=mesh,
        scratch_types=[
            pltpu.VMEM((b_per_w,), jnp.int32),
            pltpu.VMEM((b_per_w, D), jnp.float32),
            pltpu.VMEM((b_per_w, D), jnp.float32),
            pltpu.VMEM((b_per_w, D), jnp.float32),
            pltpu.VMEM((b_per_w,), jnp.float32),
            pltpu.SemaphoreType.DMA,
        ],
        compiler_params=pltpu.CompilerParams(needs_layout_passes=False),
    )
    def sc_update(mem_hbm, idx_hbm, val_hbm, upd_hbm, t_hbm,
                  idx_v, g_v, v_v, u_v, t_v, sem):
        wid = lax.axis_index("s") * _NC + lax.axis_index("c")
        base = wid * b_per_w
        pltpu.sync_copy(idx_hbm.at[pl.ds(base, b_per_w)], idx_v)
        pltpu.async_copy(mem_hbm.at[idx_v], g_v, sem).wait()
        pltpu.sync_copy(val_hbm.at[pl.ds(base, b_per_w)], v_v)

        lane = lax.broadcasted_iota(jnp.int32, (_L,), 0)

        for grp in range(b_per_w // _L):
            def row(j, tacc, _grp=grp):
                i = _grp * _L + j
                vs, gs = [], []
                acc_vv = jnp.zeros((_L,), jnp.float32)
                acc_gg = jnp.zeros((_L,), jnp.float32)
                acc_gv = jnp.zeros((_L,), jnp.float32)
                for c in range(nch):
                    vc = v_v[i, pl.ds(c * _L, _L)]
                    gc = g_v[i, pl.ds(c * _L, _L)]
                    vs.append(vc)
                    gs.append(gc)
                    acc_vv = acc_vv + vc * vc
                    acc_gg = acc_gg + gc * gc
                    acc_gv = acc_gv + gc * vc
                rv = _vrsqrt(
                    lax.broadcast_in_dim(jnp.sum(acc_vv, axis=0), (_L,), ()))
                rg = _vrsqrt(
                    lax.broadcast_in_dim(jnp.sum(acc_gg, axis=0), (_L,), ()))
                t = jnp.sum(acc_gv * rv * rg * (1.0 / TEMP), axis=0)
                tacc = jnp.where(lane == j,
                                 lax.broadcast_in_dim(t, (_L,), ()), tacc)
                us = []
                acc_uu = jnp.zeros((_L,), jnp.float32)
                for c in range(nch):
                    uc = (SMOOTH_WEIGHT * gs[c]
                          + (1.0 - SMOOTH_WEIGHT) * (vs[c] * rv))
                    us.append(uc)
                    acc_uu = acc_uu + uc * uc
                ru = _vrsqrt(
                    lax.broadcast_in_dim(jnp.sum(acc_uu, axis=0), (_L,), ()))
                for c in range(nch):
                    u_v[i, pl.ds(c * _L, _L)] = us[c] * ru
                return tacc

            t_v[pl.ds(grp * _L, _L)] = lax.fori_loop(
                0, _L, row, jnp.zeros((_L,), jnp.float32))

        pltpu.sync_copy(u_v, upd_hbm.at[pl.ds(base, b_per_w)])
        pltpu.sync_copy(t_v, t_hbm.at[pl.ds(base, b_per_w)])

    return sc_update


def _make_tc_matmul(M, D, B, Mb):
    K = M // Mb
    scale = math.log2(math.e) / TEMP

    def body(mem_ref, val_ref, s_ref, out_ref, valn):
        k = pl.program_id(0)

        @pl.when(k == 0)
        def _init():
            v = val_ref[...]
            nv = jnp.sqrt(jnp.sum(v * v, axis=1, keepdims=True))
            valn[...] = (v * (scale / (nv + 1e-12))).astype(jnp.bfloat16)
            s_ref[...] = jnp.zeros((B, 1), jnp.float32)

        g = mem_ref[...]
        out_ref[...] = g
        nrm = jnp.sqrt(jnp.sum(g * g, axis=1, keepdims=True))
        gn = (g * (1.0 / (nrm + 1e-12))).astype(jnp.bfloat16)
        l2 = lax.dot_general(
            valn[...], gn, (((1,), (1,)), ((), ())),
            preferred_element_type=jnp.float32)
        s_ref[...] += jnp.sum(jnp.exp2(l2), axis=1, keepdims=True)

    return pl.pallas_call(
        body,
        grid=(K,),
        in_specs=[
            pl.BlockSpec((Mb, D), lambda k: (k, 0)),
            pl.BlockSpec((B, D), lambda k: (0, 0)),
        ],
        out_specs=[
            pl.BlockSpec((B, 1), lambda k: (0, 0)),
            pl.BlockSpec((Mb, D), lambda k: (k, 0)),
        ],
        out_shape=[
            jax.ShapeDtypeStruct((B, 1), jnp.float32),
            jax.ShapeDtypeStruct((M, D), jnp.float32),
        ],
        scratch_shapes=[
            pltpu.VMEM((B, D), jnp.bfloat16),
        ],
        compiler_params=pltpu.CompilerParams(
            dimension_semantics=("arbitrary",)),
    )


def _make_tc_finalize(M, D, B):
    def body(idx_smem, s_ref, t_ref, upd_ref, base_any,
             loss_ref, out_any, sem):
        lse = jnp.log(s_ref[...])
        loss = jnp.mean(lse - t_ref[...])
        loss_ref[...] = jnp.full((8, 128), loss, jnp.float32)
        nchunk = 8
        ch = B // nchunk

        def chunk(c, _):
            def fire(j, _):
                i = c * ch + j
                r = idx_smem[i]
                pltpu.make_async_copy(
                    upd_ref.at[pl.ds(i, 1), :],
                    out_any.at[pl.ds(r, 1), :], sem).start()
                return 0

            lax.fori_loop(0, ch, fire, 0)

            def drain(j, _):
                pltpu.make_async_copy(
                    upd_ref.at[pl.ds(0, 1), :],
                    out_any.at[pl.ds(0, 1), :], sem).wait()
                return 0

            lax.fori_loop(0, ch, drain, 0)
            return 0

        lax.fori_loop(0, nchunk, chunk, 0)

    return pl.pallas_call(
        body,
        in_specs=[
            pl.BlockSpec(memory_space=pltpu.SMEM),
            pl.BlockSpec((B, 1), lambda: (0, 0)),
            pl.BlockSpec((B, 1), lambda: (0, 0)),
            pl.BlockSpec((B, D), lambda: (0, 0)),
            pl.BlockSpec(memory_space=pl.ANY),
        ],
        out_specs=[
            pl.BlockSpec((8, 128), lambda: (0, 0)),
            pl.BlockSpec(memory_space=pl.ANY),
        ],
        out_shape=[
            jax.ShapeDtypeStruct((8, 128), jnp.float32),
            jax.ShapeDtypeStruct((M, D), jnp.float32),
        ],
        scratch_shapes=[
            pltpu.SemaphoreType.DMA,
        ],
        input_output_aliases={4: 1},
    )


def kernel(mem, idx, val):
    M, D = mem.shape
    B = val.shape[0]
    upd, t = _make_sc_update(M, D, B)(mem, idx, val)
    s, base = _make_tc_matmul(M, D, B, 2000)(mem, val)
    loss8, mem_new = _make_tc_finalize(M, D, B)(
        idx, s, t.reshape(B, 1), upd, base)
    return loss8[0, 0], mem_new


# speedup vs baseline: 6.1227x; 1.2467x over previous
import functools
import math

import jax
import jax.numpy as jnp
from jax import lax
from jax.experimental import pallas as pl
from jax.experimental.pallas import tpu as pltpu
from jax.experimental.pallas import tpu_sc as plsc

TEMP = 0.05
SMOOTH_WEIGHT = 0.2

_NC = 2
_NS = 16
_NW = _NC * _NS
_L = 16


def _vrsqrt(x):
    i = plsc.bitcast(x, jnp.int32)
    i = jnp.int32(0x5F3759DF) - lax.shift_right_arithmetic(i, 1)
    y = plsc.bitcast(i, jnp.float32)
    for _ in range(3):
        y = y * (1.5 - 0.5 * x * y * y)
    return y


def _make_sc_update(M, D, B):
    b_per_w = B // _NW
    nch = D // _L
    mesh = plsc.VectorSubcoreMesh(core_axis_name="c", subcore_axis_name="s")

    @functools.partial(
        pl.kernel,
        out_type=(
            jax.ShapeDtypeStruct((B, D), jnp.float32),
            jax.ShapeDtypeStruct((B,), jnp.float32),
        ),
        mesh=mesh,
        scratch_types=[
            pltpu.VMEM((b_per_w,), jnp.int32),
            pltpu.VMEM((b_per_w, D), jnp.float32),
            pltpu.VMEM((b_per_w, D), jnp.float32),
            pltpu.VMEM((b_per_w, D), jnp.float32),
            pltpu.VMEM((b_per_w,), jnp.float32),
            pltpu.SemaphoreType.DMA,
        ],
        compiler_params=pltpu.CompilerParams(needs_layout_passes=False),
    )
    def sc_update(mem_hbm, idx_hbm, val_hbm, upd_hbm, t_hbm,
                  idx_v, g_v, v_v, u_v, t_v, sem):
        wid = lax.axis_index("s") * _NC + lax.axis_index("c")
        base = wid * b_per_w
        pltpu.sync_copy(idx_hbm.at[pl.ds(base, b_per_w)], idx_v)
        pltpu.async_copy(mem_hbm.at[idx_v], g_v, sem).wait()
        pltpu.sync_copy(val_hbm.at[pl.ds(base, b_per_w)], v_v)

        lane = lax.broadcasted_iota(jnp.int32, (_L,), 0)

        for grp in range(b_per_w // _L):
            def row(j, tacc, _grp=grp):
                i = _grp * _L + j
                vs, gs = [], []
                acc_vv = jnp.zeros((_L,), jnp.float32)
                acc_gg = jnp.zeros((_L,), jnp.float32)
                acc_gv = jnp.zeros((_L,), jnp.float32)
                for c in range(nch):
                    vc = v_v[i, pl.ds(c * _L, _L)]
                    gc = g_v[i, pl.ds(c * _L, _L)]
                    vs.append(vc)
                    gs.append(gc)
                    acc_vv = acc_vv + vc * vc
                    acc_gg = acc_gg + gc * gc
                    acc_gv = acc_gv + gc * vc
                rv = _vrsqrt(
                    lax.broadcast_in_dim(jnp.sum(acc_vv, axis=0), (_L,), ()))
                rg = _vrsqrt(
                    lax.broadcast_in_dim(jnp.sum(acc_gg, axis=0), (_L,), ()))
                t = jnp.sum(acc_gv * rv * rg * (1.0 / TEMP), axis=0)
                tacc = jnp.where(lane == j,
                                 lax.broadcast_in_dim(t, (_L,), ()), tacc)
                us = []
                acc_uu = jnp.zeros((_L,), jnp.float32)
                for c in range(nch):
                    uc = (SMOOTH_WEIGHT * gs[c]
                          + (1.0 - SMOOTH_WEIGHT) * (vs[c] * rv))
                    us.append(uc)
                    acc_uu = acc_uu + uc * uc
                ru = _vrsqrt(
                    lax.broadcast_in_dim(jnp.sum(acc_uu, axis=0), (_L,), ()))
                for c in range(nch):
                    u_v[i, pl.ds(c * _L, _L)] = us[c] * ru
                return tacc

            t_v[pl.ds(grp * _L, _L)] = lax.fori_loop(
                0, _L, row, jnp.zeros((_L,), jnp.float32))

        pltpu.sync_copy(u_v, upd_hbm.at[pl.ds(base, b_per_w)])
        pltpu.sync_copy(t_v, t_hbm.at[pl.ds(base, b_per_w)])

    return sc_update


def _make_tc_matmul(M, D, B, Mb):
    K = M // Mb
    scale = math.log2(math.e) / TEMP

    def body(mem_ref, val_ref, s_ref, out_ref, valn):
        k = pl.program_id(0)

        @pl.when(k == 0)
        def _init():
            v = val_ref[...]
            nv = jnp.sqrt(jnp.sum(v * v, axis=1, keepdims=True))
            valn[...] = (v * (scale / (nv + 1e-12))).astype(jnp.bfloat16)
            s_ref[...] = jnp.zeros((B, 1), jnp.float32)

        g = mem_ref[...]
        out_ref[...] = g
        h = Mb // 4
        acc = None
        for p in range(4):
            gp = g[p * h:(p + 1) * h, :]
            nsq = jnp.sum(gp * gp, axis=1, keepdims=True)
            gn = (gp * lax.rsqrt(nsq)).astype(jnp.bfloat16)
            l2 = lax.dot_general(
                valn[...], gn, (((1,), (1,)), ((), ())),
                preferred_element_type=jnp.float32)
            part = jnp.sum(jnp.exp2(l2), axis=1, keepdims=True)
            acc = part if acc is None else acc + part
        s_ref[...] += acc

    return pl.pallas_call(
        body,
        grid=(K,),
        in_specs=[
            pl.BlockSpec((Mb, D), lambda k: (k, 0)),
            pl.BlockSpec((B, D), lambda k: (0, 0)),
        ],
        out_specs=[
            pl.BlockSpec((B, 1), lambda k: (0, 0)),
            pl.BlockSpec((Mb, D), lambda k: (k, 0)),
        ],
        out_shape=[
            jax.ShapeDtypeStruct((B, 1), jnp.float32),
            jax.ShapeDtypeStruct((M, D), jnp.float32),
        ],
        scratch_shapes=[
            pltpu.VMEM((B, D), jnp.bfloat16),
        ],
        compiler_params=pltpu.CompilerParams(
            dimension_semantics=("arbitrary",)),
    )


def _make_tc_finalize(M, D, B):
    def body(idx_smem, s_ref, t_ref, upd_ref, base_any,
             loss_ref, out_any, sem):
        lse = jnp.log(s_ref[...])
        loss = jnp.mean(lse - t_ref[...])
        loss_ref[...] = jnp.full((8, 128), loss, jnp.float32)
        nchunk = 8
        ch = B // nchunk
        unroll = 8

        def chunk(c, _):
            def fire(j, _):
                base = c * ch + j * unroll
                for u in range(unroll):
                    i = base + u
                    r = idx_smem[i]
                    pltpu.make_async_copy(
                        upd_ref.at[pl.ds(i, 1), :],
                        out_any.at[pl.ds(r, 1), :], sem).start()
                return 0

            lax.fori_loop(0, ch // unroll, fire, 0)

            def drain(j, _):
                for _u in range(unroll):
                    pltpu.make_async_copy(
                        upd_ref.at[pl.ds(0, 1), :],
                        out_any.at[pl.ds(0, 1), :], sem).wait()
                return 0

            lax.fori_loop(0, ch // unroll, drain, 0)
            return 0

        lax.fori_loop(0, nchunk, chunk, 0)

    return pl.pallas_call(
        body,
        in_specs=[
            pl.BlockSpec(memory_space=pltpu.SMEM),
            pl.BlockSpec((B, 1), lambda: (0, 0)),
            pl.BlockSpec((B, 1), lambda: (0, 0)),
            pl.BlockSpec((B, D), lambda: (0, 0)),
            pl.BlockSpec(memory_space=pl.ANY),
        ],
        out_specs=[
            pl.BlockSpec((8, 128), lambda: (0, 0)),
            pl.BlockSpec(memory_space=pl.ANY),
        ],
        out_shape=[
            jax.ShapeDtypeStruct((8, 128), jnp.float32),
            jax.ShapeDtypeStruct((M, D), jnp.float32),
        ],
        scratch_shapes=[
            pltpu.SemaphoreType.DMA,
        ],
        input_output_aliases={4: 1},
    )


def kernel(mem, idx, val):
    M, D = mem.shape
    B = val.shape[0]
    upd, t = _make_sc_update(M, D, B)(mem, idx, val)
    s, base = _make_tc_matmul(M, D, B, 10000)(mem, val)
    loss8, mem_new = _make_tc_finalize(M, D, B)(
        idx, s, t.reshape(B, 1), upd, base)
    return loss8[0, 0], mem_new


# speedup vs baseline: 6.5270x; 1.0660x over previous
import functools
import math

import jax
import jax.numpy as jnp
from jax import lax
from jax.experimental import pallas as pl
from jax.experimental.pallas import tpu as pltpu
from jax.experimental.pallas import tpu_sc as plsc

TEMP = 0.05
SMOOTH_WEIGHT = 0.2

_NC = 2
_NS = 16
_NW = _NC * _NS
_L = 16


def _vrsqrt(x):
    i = plsc.bitcast(x, jnp.int32)
    i = jnp.int32(0x5F3759DF) - lax.shift_right_arithmetic(i, 1)
    y = plsc.bitcast(i, jnp.float32)
    for _ in range(3):
        y = y * (1.5 - 0.5 * x * y * y)
    return y


def _make_sc_update(M, D, B):
    b_per_w = B // _NW
    nch = D // _L
    mesh = plsc.VectorSubcoreMesh(core_axis_name="c", subcore_axis_name="s")

    @functools.partial(
        pl.kernel,
        out_type=(
            jax.ShapeDtypeStruct((B, D), jnp.float32),
            jax.ShapeDtypeStruct((B,), jnp.float32),
        ),
        mesh=mesh,
        scratch_types=[
            pltpu.VMEM((b_per_w,), jnp.int32),
            pltpu.VMEM((b_per_w, D), jnp.float32),
            pltpu.VMEM((b_per_w, D), jnp.float32),
            pltpu.VMEM((b_per_w, D), jnp.float32),
            pltpu.VMEM((b_per_w,), jnp.float32),
            pltpu.SemaphoreType.DMA,
        ],
        compiler_params=pltpu.CompilerParams(needs_layout_passes=False),
    )
    def sc_update(mem_hbm, idx_hbm, val_hbm, upd_hbm, t_hbm,
                  idx_v, g_v, v_v, u_v, t_v, sem):
        wid = lax.axis_index("s") * _NC + lax.axis_index("c")
        base = wid * b_per_w
        pltpu.sync_copy(idx_hbm.at[pl.ds(base, b_per_w)], idx_v)
        pltpu.async_copy(mem_hbm.at[idx_v], g_v, sem).wait()
        pltpu.sync_copy(val_hbm.at[pl.ds(base, b_per_w)], v_v)

        lane = lax.broadcasted_iota(jnp.int32, (_L,), 0)

        for grp in range(b_per_w // _L):
            def row(j, tacc, _grp=grp):
                i = _grp * _L + j
                vs, gs = [], []
                acc_vv = jnp.zeros((_L,), jnp.float32)
                acc_gg = jnp.zeros((_L,), jnp.float32)
                acc_gv = jnp.zeros((_L,), jnp.float32)
                for c in range(nch):
                    vc = v_v[i, pl.ds(c * _L, _L)]
                    gc = g_v[i, pl.ds(c * _L, _L)]
                    vs.append(vc)
                    gs.append(gc)
                    acc_vv = acc_vv + vc * vc
                    acc_gg = acc_gg + gc * gc
                    acc_gv = acc_gv + gc * vc
                rv = _vrsqrt(
                    lax.broadcast_in_dim(jnp.sum(acc_vv, axis=0), (_L,), ()))
                rg = _vrsqrt(
                    lax.broadcast_in_dim(jnp.sum(acc_gg, axis=0), (_L,), ()))
                t = jnp.sum(acc_gv * rv * rg * (1.0 / TEMP), axis=0)
                tacc = jnp.where(lane == j,
                                 lax.broadcast_in_dim(t, (_L,), ()), tacc)
                us = []
                acc_uu = jnp.zeros((_L,), jnp.float32)
                for c in range(nch):
                    uc = (SMOOTH_WEIGHT * gs[c]
                          + (1.0 - SMOOTH_WEIGHT) * (vs[c] * rv))
                    us.append(uc)
                    acc_uu = acc_uu + uc * uc
                ru = _vrsqrt(
                    lax.broadcast_in_dim(jnp.sum(acc_uu, axis=0), (_L,), ()))
                for c in range(nch):
                    u_v[i, pl.ds(c * _L, _L)] = us[c] * ru
                return tacc

            t_v[pl.ds(grp * _L, _L)] = lax.fori_loop(
                0, _L, row, jnp.zeros((_L,), jnp.float32))

        pltpu.sync_copy(u_v, upd_hbm.at[pl.ds(base, b_per_w)])
        pltpu.sync_copy(t_v, t_hbm.at[pl.ds(base, b_per_w)])

    return sc_update


def _make_tc_matmul(M, D, B, Mb):
    K = M // Mb
    scale = math.log2(math.e) / TEMP

    def body(mem_ref, val_ref, s_ref, out_ref, valn):
        k = pl.program_id(0)

        @pl.when(k == 0)
        def _init():
            v = val_ref[...]
            nv = jnp.sqrt(jnp.sum(v * v, axis=1, keepdims=True))
            valn[...] = (v * (scale / (nv + 1e-12))).astype(jnp.bfloat16)
            s_ref[...] = jnp.zeros((B, 1), jnp.float32)

        g = mem_ref[...]
        out_ref[...] = g
        h = Mb // 4
        acc = None
        for p in range(4):
            gp = g[p * h:(p + 1) * h, :]
            nsq = jnp.sum(gp * gp, axis=1, keepdims=True)
            gn = (gp * lax.rsqrt(nsq)).astype(jnp.bfloat16)
            l2 = lax.dot_general(
                valn[...], gn, (((1,), (1,)), ((), ())),
                preferred_element_type=jnp.float32)
            part = jnp.sum(jnp.exp2(l2), axis=1, keepdims=True)
            acc = part if acc is None else acc + part
        s_ref[...] += acc

    return pl.pallas_call(
        body,
        grid=(K,),
        in_specs=[
            pl.BlockSpec((Mb, D), lambda k: (k, 0)),
            pl.BlockSpec((B, D), lambda k: (0, 0)),
        ],
        out_specs=[
            pl.BlockSpec((B, 1), lambda k: (0, 0)),
            pl.BlockSpec((Mb, D), lambda k: (k, 0)),
        ],
        out_shape=[
            jax.ShapeDtypeStruct((B, 1), jnp.float32),
            jax.ShapeDtypeStruct((M, D), jnp.float32),
        ],
        scratch_shapes=[
            pltpu.VMEM((B, D), jnp.bfloat16),
        ],
        compiler_params=pltpu.CompilerParams(
            dimension_semantics=("arbitrary",)),
    )


def _make_tc_finalize(M, D, B):
    def body(idx_smem, s_ref, t_ref, upd_ref, base_any,
             loss_ref, out_any, sem):
        lse = jnp.log(s_ref[...])
        loss = jnp.mean(lse) - jnp.mean(t_ref[...])
        loss_ref[...] = jnp.full((1, 1), loss, jnp.float32)
        nchunk = 4
        ch = B // nchunk
        unroll = 16

        def chunk(c, _):
            def fire(j, _):
                base = c * ch + j * unroll
                for u in range(unroll):
                    i = base + u
                    r = idx_smem[i]
                    pltpu.make_async_copy(
                        upd_ref.at[pl.ds(i, 1), :],
                        out_any.at[pl.ds(r, 1), :], sem).start()
                return 0

            lax.fori_loop(0, ch // unroll, fire, 0)

            def drain(j, _):
                for _u in range(unroll):
                    pltpu.make_async_copy(
                        upd_ref.at[pl.ds(0, 1), :],
                        out_any.at[pl.ds(0, 1), :], sem).wait()
                return 0

            lax.fori_loop(0, ch // unroll, drain, 0)
            return 0

        lax.fori_loop(0, nchunk, chunk, 0)

    return pl.pallas_call(
        body,
        in_specs=[
            pl.BlockSpec(memory_space=pltpu.SMEM),
            pl.BlockSpec((B, 1), lambda: (0, 0)),
            pl.BlockSpec((B,), lambda: (0,)),
            pl.BlockSpec((B, D), lambda: (0, 0)),
            pl.BlockSpec(memory_space=pl.ANY),
        ],
        out_specs=[
            pl.BlockSpec((1, 1), lambda: (0, 0)),
            pl.BlockSpec(memory_space=pl.ANY),
        ],
        out_shape=[
            jax.ShapeDtypeStruct((1, 1), jnp.float32),
            jax.ShapeDtypeStruct((M, D), jnp.float32),
        ],
        scratch_shapes=[
            pltpu.SemaphoreType.DMA,
        ],
        input_output_aliases={4: 1},
    )


def kernel(mem, idx, val):
    M, D = mem.shape
    B = val.shape[0]
    upd, t = _make_sc_update(M, D, B)(mem, idx, val)
    s, base = _make_tc_matmul(M, D, B, 10000)(mem, val)
    loss11, mem_new = _make_tc_finalize(M, D, B)(
        idx, s, t, upd, base)
    return loss11.reshape(()), mem_new


# speedup vs baseline: 6.5785x; 1.0079x over previous
import functools
import math

import jax
import jax.numpy as jnp
from jax import lax
from jax.experimental import pallas as pl
from jax.experimental.pallas import tpu as pltpu
from jax.experimental.pallas import tpu_sc as plsc

TEMP = 0.05
SMOOTH_WEIGHT = 0.2

_NC = 2
_NS = 16
_NW = _NC * _NS
_L = 16


def _vrsqrt(x):
    i = plsc.bitcast(x, jnp.int32)
    i = jnp.int32(0x5F3759DF) - lax.shift_right_arithmetic(i, 1)
    y = plsc.bitcast(i, jnp.float32)
    for _ in range(3):
        y = y * (1.5 - 0.5 * x * y * y)
    return y


def _make_sc_update(M, D, B):
    b_per_w = B // _NW
    nch = D // _L
    mesh = plsc.VectorSubcoreMesh(core_axis_name="c", subcore_axis_name="s")

    @functools.partial(
        pl.kernel,
        out_type=(
            jax.ShapeDtypeStruct((B, D), jnp.float32),
            jax.ShapeDtypeStruct((B,), jnp.float32),
        ),
        mesh=mesh,
        scratch_types=[
            pltpu.VMEM((b_per_w,), jnp.int32),
            pltpu.VMEM((b_per_w, D), jnp.float32),
            pltpu.VMEM((b_per_w, D), jnp.float32),
            pltpu.VMEM((b_per_w, D), jnp.float32),
            pltpu.VMEM((b_per_w,), jnp.float32),
            pltpu.SemaphoreType.DMA,
        ],
        compiler_params=pltpu.CompilerParams(needs_layout_passes=False),
    )
    def sc_update(mem_hbm, idx_hbm, val_hbm, upd_hbm, t_hbm,
                  idx_v, g_v, v_v, u_v, t_v, sem):
        wid = lax.axis_index("s") * _NC + lax.axis_index("c")
        base = wid * b_per_w
        pltpu.sync_copy(idx_hbm.at[pl.ds(base, b_per_w)], idx_v)
        pltpu.async_copy(mem_hbm.at[idx_v], g_v, sem).wait()
        pltpu.sync_copy(val_hbm.at[pl.ds(base, b_per_w)], v_v)

        lane = lax.broadcasted_iota(jnp.int32, (_L,), 0)

        for grp in range(b_per_w // _L):
            def row(j, tacc, _grp=grp):
                i = _grp * _L + j
                vs, gs = [], []
                acc_vv = jnp.zeros((_L,), jnp.float32)
                acc_gg = jnp.zeros((_L,), jnp.float32)
                acc_gv = jnp.zeros((_L,), jnp.float32)
                for c in range(nch):
                    vc = v_v[i, pl.ds(c * _L, _L)]
                    gc = g_v[i, pl.ds(c * _L, _L)]
                    vs.append(vc)
                    gs.append(gc)
                    acc_vv = acc_vv + vc * vc
                    acc_gg = acc_gg + gc * gc
                    acc_gv = acc_gv + gc * vc
                rv = _vrsqrt(
                    lax.broadcast_in_dim(jnp.sum(acc_vv, axis=0), (_L,), ()))
                rg = _vrsqrt(
                    lax.broadcast_in_dim(jnp.sum(acc_gg, axis=0), (_L,), ()))
                t = jnp.sum(acc_gv * rv * rg * (1.0 / TEMP), axis=0)
                tacc = jnp.where(lane == j,
                                 lax.broadcast_in_dim(t, (_L,), ()), tacc)
                us = []
                acc_uu = jnp.zeros((_L,), jnp.float32)
                for c in range(nch):
                    uc = (SMOOTH_WEIGHT * gs[c]
                          + (1.0 - SMOOTH_WEIGHT) * (vs[c] * rv))
                    us.append(uc)
                    acc_uu = acc_uu + uc * uc
                ru = _vrsqrt(
                    lax.broadcast_in_dim(jnp.sum(acc_uu, axis=0), (_L,), ()))
                for c in range(nch):
                    u_v[i, pl.ds(c * _L, _L)] = us[c] * ru
                return tacc

            t_v[pl.ds(grp * _L, _L)] = lax.fori_loop(
                0, _L, row, jnp.zeros((_L,), jnp.float32))

        pltpu.sync_copy(u_v, upd_hbm.at[pl.ds(base, b_per_w)])
        pltpu.sync_copy(t_v, t_hbm.at[pl.ds(base, b_per_w)])

    return sc_update


def _make_tc_matmul(M, D, B, Mb):
    K = M // Mb
    scale = math.log2(math.e) / TEMP

    def body(mem_ref, val_ref, s_ref, out_any, valn, csem):
        k = pl.program_id(0)

        @pl.when(k == 0)
        def _init():
            v = val_ref[...]
            nv = jnp.sqrt(jnp.sum(v * v, axis=1, keepdims=True))
            valn[...] = (v * (scale / (nv + 1e-12))).astype(jnp.bfloat16)
            s_ref[...] = jnp.zeros((B, 1), jnp.float32)

        cp = pltpu.make_async_copy(
            mem_ref, out_any.at[pl.ds(k * Mb, Mb), :], csem)
        cp.start()
        g = mem_ref[...]
        h = Mb // 8
        acc = None
        for p in range(8):
            gp = g[p * h:(p + 1) * h, :]
            nsq = jnp.sum(gp * gp, axis=1, keepdims=True)
            gn = (gp * lax.rsqrt(nsq)).astype(jnp.bfloat16)
            l2 = lax.dot_general(
                valn[...], gn, (((1,), (1,)), ((), ())),
                preferred_element_type=jnp.float32)
            part = jnp.sum(jnp.exp2(l2), axis=1, keepdims=True)
            acc = part if acc is None else acc + part
        s_ref[...] += acc
        cp.wait()

    return pl.pallas_call(
        body,
        grid=(K,),
        in_specs=[
            pl.BlockSpec((Mb, D), lambda k: (k, 0)),
            pl.BlockSpec((B, D), lambda k: (0, 0)),
        ],
        out_specs=[
            pl.BlockSpec((B, 1), lambda k: (0, 0)),
            pl.BlockSpec(memory_space=pl.ANY),
        ],
        out_shape=[
            jax.ShapeDtypeStruct((B, 1), jnp.float32),
            jax.ShapeDtypeStruct((M, D), jnp.float32),
        ],
        scratch_shapes=[
            pltpu.VMEM((B, D), jnp.bfloat16),
            pltpu.SemaphoreType.DMA,
        ],
        compiler_params=pltpu.CompilerParams(
            dimension_semantics=("arbitrary",)),
    )


def _make_tc_finalize(M, D, B):
    def body(idx_smem, s_ref, t_ref, upd_ref, base_any,
             loss_ref, out_any, sem):
        lse = jnp.log(s_ref[...])
        loss = jnp.mean(lse) - jnp.mean(t_ref[...])
        loss_ref[...] = jnp.full((1, 1), loss, jnp.float32)
        nchunk = 4
        ch = B // nchunk
        unroll = 16

        def chunk(c, _):
            def fire(j, _):
                base = c * ch + j * unroll
                for u in range(unroll):
                    i = base + u
                    r = idx_smem[i]
                    pltpu.make_async_copy(
                        upd_ref.at[pl.ds(i, 1), :],
                        out_any.at[pl.ds(r, 1), :], sem).start()
                return 0

            lax.fori_loop(0, ch // unroll, fire, 0)

            def drain(j, _):
                for _u in range(unroll):
                    pltpu.make_async_copy(
                        upd_ref.at[pl.ds(0, 1), :],
                        out_any.at[pl.ds(0, 1), :], sem).wait()
                return 0

            lax.fori_loop(0, ch // unroll, drain, 0)
            return 0

        lax.fori_loop(0, nchunk, chunk, 0)

    return pl.pallas_call(
        body,
        in_specs=[
            pl.BlockSpec(memory_space=pltpu.SMEM),
            pl.BlockSpec((B, 1), lambda: (0, 0)),
            pl.BlockSpec((B,), lambda: (0,)),
            pl.BlockSpec((B, D), lambda: (0, 0)),
            pl.BlockSpec(memory_space=pl.ANY),
        ],
        out_specs=[
            pl.BlockSpec((1, 1), lambda: (0, 0)),
            pl.BlockSpec(memory_space=pl.ANY),
        ],
        out_shape=[
            jax.ShapeDtypeStruct((1, 1), jnp.float32),
            jax.ShapeDtypeStruct((M, D), jnp.float32),
        ],
        scratch_shapes=[
            pltpu.SemaphoreType.DMA,
        ],
        input_output_aliases={4: 1},
    )


def kernel(mem, idx, val):
    M, D = mem.shape
    B = val.shape[0]
    upd, t = _make_sc_update(M, D, B)(mem, idx, val)
    s, base = _make_tc_matmul(M, D, B, 20000)(mem, val)
    loss11, mem_new = _make_tc_finalize(M, D, B)(
        idx, s, t, upd, base)
    return loss11.reshape(()), mem_new


# speedup vs baseline: 6.6163x; 1.0057x over previous
import functools
import math

import jax
import jax.numpy as jnp
from jax import lax
from jax.experimental import pallas as pl
from jax.experimental.pallas import tpu as pltpu
from jax.experimental.pallas import tpu_sc as plsc

TEMP = 0.05
SMOOTH_WEIGHT = 0.2

_NC = 2
_NS = 16
_NW = _NC * _NS
_L = 16


def _vrsqrt(x):
    i = plsc.bitcast(x, jnp.int32)
    i = jnp.int32(0x5F3759DF) - lax.shift_right_arithmetic(i, 1)
    y = plsc.bitcast(i, jnp.float32)
    for _ in range(3):
        y = y * (1.5 - 0.5 * x * y * y)
    return y


def _make_sc_update(M, D, B):
    b_per_w = B // _NW
    nch = D // _L
    mesh = plsc.VectorSubcoreMesh(core_axis_name="c", subcore_axis_name="s")

    @functools.partial(
        pl.kernel,
        out_type=(
            jax.ShapeDtypeStruct((B, D), jnp.float32),
            jax.ShapeDtypeStruct((B,), jnp.float32),
        ),
        mesh=mesh,
        scratch_types=[
            pltpu.VMEM((b_per_w,), jnp.int32),
            pltpu.VMEM((b_per_w, D), jnp.float32),
            pltpu.VMEM((b_per_w, D), jnp.float32),
            pltpu.VMEM((b_per_w, D), jnp.float32),
            pltpu.VMEM((b_per_w,), jnp.float32),
            pltpu.SemaphoreType.DMA,
        ],
        compiler_params=pltpu.CompilerParams(needs_layout_passes=False),
    )
    def sc_update(mem_hbm, idx_hbm, val_hbm, upd_hbm, t_hbm,
                  idx_v, g_v, v_v, u_v, t_v, sem):
        wid = lax.axis_index("s") * _NC + lax.axis_index("c")
        base = wid * b_per_w
        pltpu.sync_copy(idx_hbm.at[pl.ds(base, b_per_w)], idx_v)
        pltpu.async_copy(mem_hbm.at[idx_v], g_v, sem).wait()
        pltpu.sync_copy(val_hbm.at[pl.ds(base, b_per_w)], v_v)

        lane = lax.broadcasted_iota(jnp.int32, (_L,), 0)

        for grp in range(b_per_w // _L):
            def row(j, tacc, _grp=grp):
                i = _grp * _L + j
                vs, gs = [], []
                acc_vv = jnp.zeros((_L,), jnp.float32)
                acc_gg = jnp.zeros((_L,), jnp.float32)
                acc_gv = jnp.zeros((_L,), jnp.float32)
                for c in range(nch):
                    vc = v_v[i, pl.ds(c * _L, _L)]
                    gc = g_v[i, pl.ds(c * _L, _L)]
                    vs.append(vc)
                    gs.append(gc)
                    acc_vv = acc_vv + vc * vc
                    acc_gg = acc_gg + gc * gc
                    acc_gv = acc_gv + gc * vc
                rv = _vrsqrt(
                    lax.broadcast_in_dim(jnp.sum(acc_vv, axis=0), (_L,), ()))
                rg = _vrsqrt(
                    lax.broadcast_in_dim(jnp.sum(acc_gg, axis=0), (_L,), ()))
                t = jnp.sum(acc_gv * rv * rg * (1.0 / TEMP), axis=0)
                tacc = jnp.where(lane == j,
                                 lax.broadcast_in_dim(t, (_L,), ()), tacc)
                us = []
                acc_uu = jnp.zeros((_L,), jnp.float32)
                for c in range(nch):
                    uc = (SMOOTH_WEIGHT * gs[c]
                          + (1.0 - SMOOTH_WEIGHT) * (vs[c] * rv))
                    us.append(uc)
                    acc_uu = acc_uu + uc * uc
                ru = _vrsqrt(
                    lax.broadcast_in_dim(jnp.sum(acc_uu, axis=0), (_L,), ()))
                for c in range(nch):
                    u_v[i, pl.ds(c * _L, _L)] = us[c] * ru
                return tacc

            t_v[pl.ds(grp * _L, _L)] = lax.fori_loop(
                0, _L, row, jnp.zeros((_L,), jnp.float32))

        pltpu.sync_copy(u_v, upd_hbm.at[pl.ds(base, b_per_w)])
        pltpu.sync_copy(t_v, t_hbm.at[pl.ds(base, b_per_w)])

    return sc_update


def _make_tc_matmul(M, D, B, Mb):
    K = M // Mb
    scale = math.log2(math.e) / TEMP

    def body(mem_ref, val_ref, s_ref, out_any, valn, csem):
        k = pl.program_id(0)

        @pl.when(k == 0)
        def _init():
            v = val_ref[...]
            nv = jnp.sqrt(jnp.sum(v * v, axis=1, keepdims=True))
            valn[...] = (v * (scale / (nv + 1e-12))).astype(jnp.bfloat16)
            s_ref[...] = jnp.zeros((B, 1), jnp.float32)

        cp = pltpu.make_async_copy(
            mem_ref, out_any.at[pl.ds(k * Mb, Mb), :], csem)
        cp.start()
        g = mem_ref[...]
        h = Mb // 8
        acc = None
        for p in range(8):
            gp = g[p * h:(p + 1) * h, :]
            nsq = jnp.sum(gp * gp, axis=1, keepdims=True)
            gn = (gp * lax.rsqrt(nsq)).astype(jnp.bfloat16)
            l2 = lax.dot_general(
                valn[...], gn, (((1,), (1,)), ((), ())),
                preferred_element_type=jnp.float32)
            part = jnp.sum(jnp.exp2(l2), axis=1, keepdims=True)
            acc = part if acc is None else acc + part
        s_ref[...] += acc
        cp.wait()

    return pl.pallas_call(
        body,
        grid=(K,),
        in_specs=[
            pl.BlockSpec((Mb, D), lambda k: (k, 0)),
            pl.BlockSpec((B, D), lambda k: (0, 0)),
        ],
        out_specs=[
            pl.BlockSpec((B, 1), lambda k: (0, 0)),
            pl.BlockSpec(memory_space=pl.ANY),
        ],
        out_shape=[
            jax.ShapeDtypeStruct((B, 1), jnp.float32),
            jax.ShapeDtypeStruct((M, D), jnp.float32),
        ],
        scratch_shapes=[
            pltpu.VMEM((B, D), jnp.bfloat16),
            pltpu.SemaphoreType.DMA,
        ],
        compiler_params=pltpu.CompilerParams(
            dimension_semantics=("arbitrary",)),
    )


def _make_tc_finalize(M, D, B):
    def body(idx_smem, s_ref, t_ref, upd_ref, base_any,
             loss_ref, out_any, sem):
        lse = jnp.log(s_ref[...])
        loss = jnp.mean(lse) - jnp.mean(t_ref[...])
        loss_ref[...] = jnp.full((1, 1), loss, jnp.float32)
        nchunk = 4
        ch = B // nchunk

        for c in range(nchunk):
            for i in range(c * ch, (c + 1) * ch):
                r = idx_smem[i]
                pltpu.make_async_copy(
                    upd_ref.at[pl.ds(i, 1), :],
                    out_any.at[pl.ds(r, 1), :], sem).start()

            def drain(j, _):
                for _u in range(16):
                    pltpu.make_async_copy(
                        upd_ref.at[pl.ds(0, 1), :],
                        out_any.at[pl.ds(0, 1), :], sem).wait()
                return 0

            lax.fori_loop(0, ch // 16, drain, 0)

    return pl.pallas_call(
        body,
        in_specs=[
            pl.BlockSpec(memory_space=pltpu.SMEM),
            pl.BlockSpec((B, 1), lambda: (0, 0)),
            pl.BlockSpec((B,), lambda: (0,)),
            pl.BlockSpec((B, D), lambda: (0, 0)),
            pl.BlockSpec(memory_space=pl.ANY),
        ],
        out_specs=[
            pl.BlockSpec((1, 1), lambda: (0, 0)),
            pl.BlockSpec(memory_space=pl.ANY),
        ],
        out_shape=[
            jax.ShapeDtypeStruct((1, 1), jnp.float32),
            jax.ShapeDtypeStruct((M, D), jnp.float32),
        ],
        scratch_shapes=[
            pltpu.SemaphoreType.DMA,
        ],
        input_output_aliases={4: 1},
    )


def kernel(mem, idx, val):
    M, D = mem.shape
    B = val.shape[0]
    upd, t = _make_sc_update(M, D, B)(mem, idx, val)
    s, base = _make_tc_matmul(M, D, B, 20000)(mem, val)
    loss11, mem_new = _make_tc_finalize(M, D, B)(
        idx, s, t, upd, base)
    return loss11.reshape(()), mem_new
